# baseline (device time: 207723 ns/iter reference)
import jax
import jax.numpy as jnp
from jax import lax
from jax.experimental import pallas as pl
from jax.experimental.pallas import tpu as pltpu

N_DEV = 8
N_HOP = N_DEV - 1
N_SLOT = 4
Q = 4
_GELU_C = 0.7978845608028654


def _gelu(y):
    return 0.5 * y * (1.0 + jnp.tanh(_GELU_C * (y + 0.044715 * y * y * y)))


def kernel(x, w_mat):
    m_per, k = x.shape
    _, n_per = w_mat.shape
    half = m_per // 2
    sub = half // Q

    xb = x.astype(jnp.bfloat16)
    wb = w_mat.astype(jnp.bfloat16)

    def body(x_ref, w_ref, out_ref, fwd, bwd, fs_sem, fr_sem, bs_sem, br_sem):
        my = lax.axis_index("i")
        right = lax.rem(my + 1, N_DEV)
        left = lax.rem(my + N_DEV - 1, N_DEV)

        barrier = pltpu.get_barrier_semaphore()
        for nbr in (left, right):
            pl.semaphore_signal(
                barrier, inc=1, device_id=(nbr,),
                device_id_type=pl.DeviceIdType.MESH,
            )
        pl.semaphore_wait(barrier, 2)

        def make(h, q, first=False):
            s, r = h % N_SLOT, (h + 1) % N_SLOT
            rows = pl.ds(q * sub, sub)
            if first:
                f_src = x_ref.at[rows, :]
                b_src = x_ref.at[pl.ds(half + q * sub, sub), :]
            else:
                f_src = fwd.at[s, rows, :]
                b_src = bwd.at[s, rows, :]
            f = pltpu.make_async_remote_copy(
                src_ref=f_src,
                dst_ref=fwd.at[r, rows, :],
                send_sem=fs_sem.at[s, q],
                recv_sem=fr_sem.at[r, q],
                device_id=(right,),
                device_id_type=pl.DeviceIdType.MESH,
            )
            b = pltpu.make_async_remote_copy(
                src_ref=b_src,
                dst_ref=bwd.at[r, rows, :],
                send_sem=bs_sem.at[s, q],
                recv_sem=br_sem.at[r, q],
                device_id=(left,),
                device_id_type=pl.DeviceIdType.MESH,
            )
            return f, b

        descs = {}
        for q in range(Q):
            f, b = make(0, q, first=True)
            f.start()
            b.start()
            descs[q] = (f, b)

        out_ref[pl.ds(my * m_per, m_per), :] = _gelu(
            jnp.dot(x_ref[:, :], w_ref[:, :], preferred_element_type=jnp.float32)
        )

        for h in range(1, N_HOP + 1):
            s = h % N_SLOT
            if h < N_HOP:
                nxt = {}
                for q in range(Q):
                    descs[q][0].wait()
                    descs[q][1].wait()
                    f, b = make(h, q)
                    f.start()
                    b.start()
                    nxt[q] = (f, b)
            else:
                for q in range(Q):
                    descs[q][0].wait()
                    descs[q][1].wait()
                nxt = None
            origin_f = lax.rem(my + N_DEV - h, N_DEV)
            origin_b = lax.rem(my + h, N_DEV)
            out_ref[pl.ds(origin_f * m_per, half), :] = _gelu(
                jnp.dot(fwd[s], w_ref[:, :], preferred_element_type=jnp.float32)
            )
            out_ref[pl.ds(origin_b * m_per + half, half), :] = _gelu(
                jnp.dot(bwd[s], w_ref[:, :], preferred_element_type=jnp.float32)
            )
            descs = nxt

    return pl.pallas_call(
        body,
        out_shape=jax.ShapeDtypeStruct((N_DEV * m_per, n_per), jnp.float32),
        in_specs=[
            pl.BlockSpec(memory_space=pltpu.VMEM),
            pl.BlockSpec(memory_space=pltpu.VMEM),
        ],
        out_specs=pl.BlockSpec(memory_space=pltpu.VMEM),
        scratch_shapes=[
            pltpu.VMEM((N_SLOT, half, k), jnp.bfloat16),
            pltpu.VMEM((N_SLOT, half, k), jnp.bfloat16),
            pltpu.SemaphoreType.DMA((N_SLOT, Q)),
            pltpu.SemaphoreType.DMA((N_SLOT, Q)),
            pltpu.SemaphoreType.DMA((N_SLOT, Q)),
            pltpu.SemaphoreType.DMA((N_SLOT, Q)),
        ],
        compiler_params=pltpu.CompilerParams(
            collective_id=0, vmem_limit_bytes=100 * 1024 * 1024
        ),
    )(xb, wb)


# device time: 204906 ns/iter; 1.0137x vs baseline; 1.0137x over previous
import jax
import jax.numpy as jnp
from jax import lax
from jax.experimental import pallas as pl
from jax.experimental.pallas import tpu as pltpu

jax.config.update("jax_compilation_cache_dir", "/tmp/jax_comp_cache")
jax.config.update("jax_persistent_cache_min_compile_time_secs", 0)

N_DEV = 8
N_HOP = N_DEV - 1
N_SLOT = 4
Q = 2
_GELU_C = 0.7978845608028654


def _gelu(y):
    return 0.5 * y * (1.0 + jnp.tanh(_GELU_C * (y + 0.044715 * y * y * y)))


def kernel(x, w_mat):
    m_per, k = x.shape
    _, n_per = w_mat.shape
    half = m_per // 2
    sub = half // Q

    xb = x.astype(jnp.bfloat16)
    wb = w_mat.astype(jnp.bfloat16)

    def body(x_ref, w_ref, out_ref, fwd, bwd, fs_sem, fr_sem, bs_sem, br_sem):
        my = lax.axis_index("i")
        right = lax.rem(my + 1, N_DEV)
        left = lax.rem(my + N_DEV - 1, N_DEV)

        barrier = pltpu.get_barrier_semaphore()
        for nbr in (left, right):
            pl.semaphore_signal(
                barrier, inc=1, device_id=(nbr,),
                device_id_type=pl.DeviceIdType.MESH,
            )
        pl.semaphore_wait(barrier, 2)

        def make(h, q, first=False):
            s, r = h % N_SLOT, (h + 1) % N_SLOT
            rows = pl.ds(q * sub, sub)
            if first:
                f_src = x_ref.at[rows, :]
                b_src = x_ref.at[pl.ds(half + q * sub, sub), :]
            else:
                f_src = fwd.at[s, rows, :]
                b_src = bwd.at[s, rows, :]
            f = pltpu.make_async_remote_copy(
                src_ref=f_src,
                dst_ref=fwd.at[r, rows, :],
                send_sem=fs_sem.at[s, q],
                recv_sem=fr_sem.at[r, q],
                device_id=(right,),
                device_id_type=pl.DeviceIdType.MESH,
            )
            b = pltpu.make_async_remote_copy(
                src_ref=b_src,
                dst_ref=bwd.at[r, rows, :],
                send_sem=bs_sem.at[s, q],
                recv_sem=br_sem.at[r, q],
                device_id=(left,),
                device_id_type=pl.DeviceIdType.MESH,
            )
            return f, b

        descs = {}
        for q in range(Q):
            f, b = make(0, q, first=True)
            f.start()
            b.start()
            descs[q] = (f, b)

        out_ref[pl.ds(my * m_per, m_per), :] = _gelu(
            jnp.dot(x_ref[:, :], w_ref[:, :], preferred_element_type=jnp.float32)
        )

        for h in range(1, N_HOP + 1):
            s = h % N_SLOT
            origin_f = lax.rem(my + N_DEV - h, N_DEV)
            origin_b = lax.rem(my + h, N_DEV)
            nxt = {} if h < N_HOP else None
            for q in range(Q):
                descs[q][0].wait()
                descs[q][1].wait()
                if h < N_HOP:
                    f, b = make(h, q)
                    f.start()
                    b.start()
                    nxt[q] = (f, b)
                f_sub = fwd[s][q * sub:(q + 1) * sub, :]
                b_sub = bwd[s][q * sub:(q + 1) * sub, :]
                out_ref[pl.ds(origin_f * m_per + q * sub, sub), :] = _gelu(
                    jnp.dot(
                        f_sub, w_ref[:, :], preferred_element_type=jnp.float32
                    )
                )
                out_ref[pl.ds(origin_b * m_per + half + q * sub, sub), :] = _gelu(
                    jnp.dot(
                        b_sub, w_ref[:, :], preferred_element_type=jnp.float32
                    )
                )
            descs = nxt

    return pl.pallas_call(
        body,
        out_shape=jax.ShapeDtypeStruct((N_DEV * m_per, n_per), jnp.float32),
        in_specs=[
            pl.BlockSpec(memory_space=pltpu.VMEM),
            pl.BlockSpec(memory_space=pltpu.VMEM),
        ],
        out_specs=pl.BlockSpec(memory_space=pltpu.VMEM),
        scratch_shapes=[
            pltpu.VMEM((N_SLOT, half, k), jnp.bfloat16),
            pltpu.VMEM((N_SLOT, half, k), jnp.bfloat16),
            pltpu.SemaphoreType.DMA((N_SLOT, Q)),
            pltpu.SemaphoreType.DMA((N_SLOT, Q)),
            pltpu.SemaphoreType.DMA((N_SLOT, Q)),
            pltpu.SemaphoreType.DMA((N_SLOT, Q)),
        ],
        compiler_params=pltpu.CompilerParams(
            collective_id=0, vmem_limit_bytes=100 * 1024 * 1024
        ),
    )(xb, wb)


# device time: 196745 ns/iter; 1.0558x vs baseline; 1.0415x over previous
import jax
import jax.numpy as jnp
from jax import lax
from jax.experimental import pallas as pl
from jax.experimental.pallas import tpu as pltpu

jax.config.update("jax_compilation_cache_dir", "/tmp/jax_comp_cache")
jax.config.update("jax_persistent_cache_min_compile_time_secs", 0)

N_DEV = 8
N_HOP = N_DEV - 1
N_SLOT = 4
Q = 2
_GELU_C = 0.7978845608028654


def _gelu(y):
    return 0.5 * y * (1.0 + jnp.tanh(_GELU_C * (y + 0.044715 * y * y * y)))


def kernel(x, w_mat):
    m_per, k = x.shape
    _, n_per = w_mat.shape
    half = m_per // 2
    sub = half // Q

    xb = x.astype(jnp.bfloat16)
    wb = w_mat.astype(jnp.bfloat16)

    def body(
        x_hbm, w_hbm, out_hbm, xv, wv, stage, fwd, bwd,
        ld_sem, st_sem, fs_sem, fr_sem, bs_sem, br_sem,
    ):
        my = lax.axis_index("i")
        right = lax.rem(my + 1, N_DEV)
        left = lax.rem(my + N_DEV - 1, N_DEV)

        x_load = pltpu.make_async_copy(x_hbm, xv, ld_sem.at[0])
        w_load = pltpu.make_async_copy(w_hbm, wv, ld_sem.at[1])
        x_load.start()
        w_load.start()

        barrier = pltpu.get_barrier_semaphore()
        for nbr in (left, right):
            pl.semaphore_signal(
                barrier, inc=1, device_id=(nbr,),
                device_id_type=pl.DeviceIdType.MESH,
            )
        pl.semaphore_wait(barrier, 2)

        def make(h, q, first=False):
            s, r = h % N_SLOT, (h + 1) % N_SLOT
            rows = pl.ds(q * sub, sub)
            if first:
                f_src = xv.at[rows, :]
                b_src = xv.at[pl.ds(half + q * sub, sub), :]
            else:
                f_src = fwd.at[s, rows, :]
                b_src = bwd.at[s, rows, :]
            f = pltpu.make_async_remote_copy(
                src_ref=f_src,
                dst_ref=fwd.at[r, rows, :],
                send_sem=fs_sem.at[s, q],
                recv_sem=fr_sem.at[r, q],
                device_id=(right,),
                device_id_type=pl.DeviceIdType.MESH,
            )
            b = pltpu.make_async_remote_copy(
                src_ref=b_src,
                dst_ref=bwd.at[r, rows, :],
                send_sem=bs_sem.at[s, q],
                recv_sem=br_sem.at[r, q],
                device_id=(left,),
                device_id_type=pl.DeviceIdType.MESH,
            )
            return f, b

        x_load.wait()
        descs = {}
        for q in range(Q):
            f, b = make(0, q, first=True)
            f.start()
            b.start()
            descs[q] = (f, b)

        pending = [None, None]
        counter = [0]

        def emit(row_start, val):
            i = counter[0] % 2
            if pending[i] is not None:
                pending[i].wait()
            stage[i, :, :] = val
            c = pltpu.make_async_copy(
                stage.at[i], out_hbm.at[pl.ds(row_start, sub), :], st_sem.at[i]
            )
            c.start()
            pending[i] = c
            counter[0] += 1

        w_load.wait()
        for q in range(2 * Q):
            emit(
                my * m_per + q * sub,
                _gelu(
                    jnp.dot(
                        xv[q * sub:(q + 1) * sub, :], wv[:, :],
                        preferred_element_type=jnp.float32,
                    )
                ),
            )

        for h in range(1, N_HOP + 1):
            s = h % N_SLOT
            origin_f = lax.rem(my + N_DEV - h, N_DEV)
            origin_b = lax.rem(my + h, N_DEV)
            nxt = {} if h < N_HOP else None
            for q in range(Q):
                descs[q][0].wait()
                descs[q][1].wait()
                if h < N_HOP:
                    f, b = make(h, q)
                    f.start()
                    b.start()
                    nxt[q] = (f, b)
                f_sub = fwd[s][q * sub:(q + 1) * sub, :]
                b_sub = bwd[s][q * sub:(q + 1) * sub, :]
                emit(
                    origin_f * m_per + q * sub,
                    _gelu(
                        jnp.dot(
                            f_sub, wv[:, :], preferred_element_type=jnp.float32
                        )
                    ),
                )
                emit(
                    origin_b * m_per + half + q * sub,
                    _gelu(
                        jnp.dot(
                            b_sub, wv[:, :], preferred_element_type=jnp.float32
                        )
                    ),
                )
            descs = nxt

        for p in pending:
            if p is not None:
                p.wait()

    return pl.pallas_call(
        body,
        out_shape=jax.ShapeDtypeStruct((N_DEV * m_per, n_per), jnp.float32),
        in_specs=[
            pl.BlockSpec(memory_space=pl.ANY),
            pl.BlockSpec(memory_space=pl.ANY),
        ],
        out_specs=pl.BlockSpec(memory_space=pl.ANY),
        scratch_shapes=[
            pltpu.VMEM((m_per, k), jnp.bfloat16),
            pltpu.VMEM((k, n_per), jnp.bfloat16),
            pltpu.VMEM((2, sub, n_per), jnp.float32),
            pltpu.VMEM((N_SLOT, half, k), jnp.bfloat16),
            pltpu.VMEM((N_SLOT, half, k), jnp.bfloat16),
            pltpu.SemaphoreType.DMA((2,)),
            pltpu.SemaphoreType.DMA((2,)),
            pltpu.SemaphoreType.DMA((N_SLOT, Q)),
            pltpu.SemaphoreType.DMA((N_SLOT, Q)),
            pltpu.SemaphoreType.DMA((N_SLOT, Q)),
            pltpu.SemaphoreType.DMA((N_SLOT, Q)),
        ],
        compiler_params=pltpu.CompilerParams(
            collective_id=0, vmem_limit_bytes=100 * 1024 * 1024
        ),
    )(xb, wb)


# device time: 186814 ns/iter; 1.1119x vs baseline; 1.0532x over previous
import jax
import jax.numpy as jnp
from jax import lax
from jax.experimental import pallas as pl
from jax.experimental.pallas import tpu as pltpu

jax.config.update("jax_compilation_cache_dir", "/tmp/jax_comp_cache")
jax.config.update("jax_persistent_cache_min_compile_time_secs", 0)

N_DEV = 8
N_HOP = N_DEV - 1
N_SLOT = 4
Q = 2
_GELU_C = 0.7978845608028654


def _gelu(y):
    return 0.5 * y * (1.0 + jnp.tanh(_GELU_C * (y + 0.044715 * y * y * y)))


def kernel(x, w_mat):
    m_per, k = x.shape
    _, n_per = w_mat.shape
    half = m_per // 2
    sub = half // Q

    xb = x.astype(jnp.bfloat16)
    kc = k // 4

    def body(
        x_hbm, w_hbm, out_hbm, xv, wv, wstage, stage, fwd, bwd,
        ld_sem, st_sem, fs_sem, fr_sem, bs_sem, br_sem,
    ):
        my = lax.axis_index("i")
        right = lax.rem(my + 1, N_DEV)
        left = lax.rem(my + N_DEV - 1, N_DEV)

        x_load = pltpu.make_async_copy(x_hbm, xv, ld_sem.at[0])
        x_load.start()

        def w_load(c):
            return pltpu.make_async_copy(
                w_hbm.at[pl.ds(c * kc, kc), :],
                wstage.at[c % 2],
                ld_sem.at[1 + c % 2],
            )

        w_loads = {c: w_load(c) for c in range(4)}
        w_loads[0].start()
        w_loads[1].start()

        barrier = pltpu.get_barrier_semaphore()
        for nbr in (left, right):
            pl.semaphore_signal(
                barrier, inc=1, device_id=(nbr,),
                device_id_type=pl.DeviceIdType.MESH,
            )
        pl.semaphore_wait(barrier, 2)

        def make(h, q, first=False):
            s, r = h % N_SLOT, (h + 1) % N_SLOT
            rows = pl.ds(q * sub, sub)
            if first:
                f_src = xv.at[rows, :]
                b_src = xv.at[pl.ds(half + q * sub, sub), :]
            else:
                f_src = fwd.at[s, rows, :]
                b_src = bwd.at[s, rows, :]
            f = pltpu.make_async_remote_copy(
                src_ref=f_src,
                dst_ref=fwd.at[r, rows, :],
                send_sem=fs_sem.at[s, q],
                recv_sem=fr_sem.at[r, q],
                device_id=(right,),
                device_id_type=pl.DeviceIdType.MESH,
            )
            b = pltpu.make_async_remote_copy(
                src_ref=b_src,
                dst_ref=bwd.at[r, rows, :],
                send_sem=bs_sem.at[s, q],
                recv_sem=br_sem.at[r, q],
                device_id=(left,),
                device_id_type=pl.DeviceIdType.MESH,
            )
            return f, b

        x_load.wait()
        descs = {}
        for q in range(Q):
            f, b = make(0, q, first=True)
            f.start()
            b.start()
            descs[q] = (f, b)

        pending = [None, None]
        counter = [0]

        def emit(row_start, val):
            i = counter[0] % 2
            if pending[i] is not None:
                pending[i].wait()
            stage[i, :, :] = val
            c = pltpu.make_async_copy(
                stage.at[i], out_hbm.at[pl.ds(row_start, sub), :], st_sem.at[i]
            )
            c.start()
            pending[i] = c
            counter[0] += 1

        for c in range(4):
            w_loads[c].wait()
            wv[pl.ds(c * kc, kc), :] = wstage[c % 2].astype(jnp.bfloat16)
            if c + 2 < 4:
                w_loads[c + 2].start()

        for q in range(2 * Q):
            emit(
                my * m_per + q * sub,
                _gelu(
                    jnp.dot(
                        xv[q * sub:(q + 1) * sub, :], wv[:, :],
                        preferred_element_type=jnp.float32,
                    )
                ),
            )

        for h in range(1, N_HOP + 1):
            s = h % N_SLOT
            origin_f = lax.rem(my + N_DEV - h, N_DEV)
            origin_b = lax.rem(my + h, N_DEV)
            nxt = {} if h < N_HOP else None
            for q in range(Q):
                descs[q][0].wait()
                descs[q][1].wait()
                if h < N_HOP:
                    f, b = make(h, q)
                    f.start()
                    b.start()
                    nxt[q] = (f, b)
                f_sub = fwd[s][q * sub:(q + 1) * sub, :]
                b_sub = bwd[s][q * sub:(q + 1) * sub, :]
                emit(
                    origin_f * m_per + q * sub,
                    _gelu(
                        jnp.dot(
                            f_sub, wv[:, :], preferred_element_type=jnp.float32
                        )
                    ),
                )
                emit(
                    origin_b * m_per + half + q * sub,
                    _gelu(
                        jnp.dot(
                            b_sub, wv[:, :], preferred_element_type=jnp.float32
                        )
                    ),
                )
            descs = nxt

        for p in pending:
            if p is not None:
                p.wait()

    return pl.pallas_call(
        body,
        out_shape=jax.ShapeDtypeStruct((N_DEV * m_per, n_per), jnp.float32),
        in_specs=[
            pl.BlockSpec(memory_space=pl.ANY),
            pl.BlockSpec(memory_space=pl.ANY),
        ],
        out_specs=pl.BlockSpec(memory_space=pl.ANY),
        scratch_shapes=[
            pltpu.VMEM((m_per, k), jnp.bfloat16),
            pltpu.VMEM((k, n_per), jnp.bfloat16),
            pltpu.VMEM((2, k // 4, n_per), jnp.float32),
            pltpu.VMEM((2, sub, n_per), jnp.float32),
            pltpu.VMEM((N_SLOT, half, k), jnp.bfloat16),
            pltpu.VMEM((N_SLOT, half, k), jnp.bfloat16),
            pltpu.SemaphoreType.DMA((3,)),
            pltpu.SemaphoreType.DMA((2,)),
            pltpu.SemaphoreType.DMA((N_SLOT, Q)),
            pltpu.SemaphoreType.DMA((N_SLOT, Q)),
            pltpu.SemaphoreType.DMA((N_SLOT, Q)),
            pltpu.SemaphoreType.DMA((N_SLOT, Q)),
        ],
        compiler_params=pltpu.CompilerParams(
            collective_id=0, vmem_limit_bytes=100 * 1024 * 1024
        ),
    )(xb, w_mat)
